# baseline (speedup 1.0000x reference)
import functools

import jax
import jax.numpy as jnp
from jax import lax
from jax.experimental import pallas as pl
from jax.experimental.pallas import tpu as pltpu


def _pass1(x_ref, cf_ref, watt_ref, wa_ref, wb_ref, wc_ref, wn_ref, wl_ref,
           wd1_ref, wd2_ref, d_ref, s_ref, q_ref, *, CN, CL):
    f32 = jnp.float32
    xb = x_ref[0]
    cf = cf_ref[0]
    J = CN * CL

    S = (lax.broadcasted_iota(jnp.int32, (J, CN), 0) // CL
         == lax.broadcasted_iota(jnp.int32, (J, CN), 1)).astype(f32)
    S2 = (lax.broadcasted_iota(jnp.int32, (CN, J), 1) // CL
          == lax.broadcasted_iota(jnp.int32, (CN, J), 0)).astype(f32)
    T = (lax.broadcasted_iota(jnp.int32, (J, CL), 0) % CL
         == lax.broadcasted_iota(jnp.int32, (J, CL), 1)).astype(f32)
    T2 = (lax.broadcasted_iota(jnp.int32, (CL, J), 1) % CL
          == lax.broadcasted_iota(jnp.int32, (CL, J), 0)).astype(f32)

    att = jnp.dot(watt_ref[...], cf, preferred_element_type=f32, precision=lax.Precision.HIGHEST)
    e = jnp.exp(att - jnp.max(att))
    den_k = jnp.dot(e, S, preferred_element_type=f32, precision=lax.Precision.HIGHEST)
    den_l = jnp.dot(e, T, preferred_element_type=f32, precision=lax.Precision.HIGHEST)
    soft_last = e / jnp.dot(den_k, S2, preferred_element_type=f32, precision=lax.Precision.HIGHEST)
    soft_pen = e / jnp.dot(den_l, T2, preferred_element_type=f32, precision=lax.Precision.HIGHEST)

    curver_inter = jnp.dot(cf * soft_last, S, preferred_element_type=f32, precision=lax.Precision.HIGHEST)
    curves_intra = jnp.dot(cf * soft_pen, T, preferred_element_type=f32, precision=lax.Precision.HIGHEST)

    CI = jnp.dot(wa_ref[...], curver_inter, preferred_element_type=f32, precision=lax.Precision.HIGHEST)
    CLm = jnp.dot(wb_ref[...], curves_intra, preferred_element_type=f32, precision=lax.Precision.HIGHEST)
    WnCI = jnp.dot(wn_ref[...], CI, preferred_element_type=f32, precision=lax.Precision.HIGHEST)
    WlCL = jnp.dot(wl_ref[...], CLm, preferred_element_type=f32, precision=lax.Precision.HIGHEST)

    G1 = lax.dot_general(CI, wc_ref[...], (((0,), (0,)), ((), ())),
                         preferred_element_type=f32, precision=lax.Precision.HIGHEST)
    G2 = lax.dot_general(CLm, wc_ref[...], (((0,), (0,)), ((), ())),
                         preferred_element_type=f32, precision=lax.Precision.HIGHEST)
    M1 = jnp.dot(wd1_ref[...], WnCI, preferred_element_type=f32, precision=lax.Precision.HIGHEST)
    M2 = jnp.dot(wd2_ref[...], WlCL, preferred_element_type=f32, precision=lax.Precision.HIGHEST)

    Li = jnp.dot(G1, xb, preferred_element_type=f32, precision=lax.Precision.HIGHEST)
    Ei = jnp.exp(Li - jnp.max(Li, axis=0, keepdims=True))
    Pi = Ei / jnp.sum(Ei, axis=0, keepdims=True)

    Ll = jnp.dot(G2, xb, preferred_element_type=f32, precision=lax.Precision.HIGHEST)
    El = jnp.exp(Ll - jnp.max(Ll, axis=0, keepdims=True))
    Pl = El / jnp.sum(El, axis=0, keepdims=True)

    db = (jnp.dot(M1, Pi, preferred_element_type=f32, precision=lax.Precision.HIGHEST)
          + jnp.dot(M2, Pl, preferred_element_type=f32, precision=lax.Precision.HIGHEST))
    d_ref[0] = db.astype(jnp.bfloat16)
    s_ref[0] = jnp.sum(db, axis=1, keepdims=True)
    q_ref[0] = jnp.sum(db * db, axis=1, keepdims=True)


def _pass2(d_ref, x_ref, s_ref, q_ref, g_ref, b_ref, out_ref, *, count):
    mean = jnp.sum(s_ref[...], axis=1, keepdims=True) / count
    var = jnp.sum(q_ref[...], axis=1, keepdims=True) / count - mean * mean
    scale = g_ref[...] * lax.rsqrt(var + 1e-5)
    shift = b_ref[...] - mean * scale
    y = x_ref[0] + d_ref[0].astype(jnp.float32) * scale + shift
    out_ref[0] = jnp.where(y >= 0, y, 0.2 * y)


@jax.jit
def kernel(x, curves, w_att, Wa, Wb, Wc, Wn, Wl, Wd, gamma, beta):
    B, C, N = x.shape
    CN, CL = curves.shape[2], curves.shape[3]
    MID = Wa.shape[0]
    J = CN * CL
    f32 = jnp.float32

    curves_flat = curves.reshape(B, C, J)
    watt2 = w_att.reshape(1, C)
    Wd1 = Wd[:, :MID]
    Wd2 = Wd[:, MID:]

    d, s, q = pl.pallas_call(
        functools.partial(_pass1, CN=CN, CL=CL),
        grid=(B,),
        in_specs=[
            pl.BlockSpec((1, C, N), lambda b: (b, 0, 0)),
            pl.BlockSpec((1, C, J), lambda b: (b, 0, 0)),
            pl.BlockSpec((1, C), lambda b: (0, 0)),
            pl.BlockSpec((MID, C), lambda b: (0, 0)),
            pl.BlockSpec((MID, C), lambda b: (0, 0)),
            pl.BlockSpec((MID, C), lambda b: (0, 0)),
            pl.BlockSpec((MID, MID), lambda b: (0, 0)),
            pl.BlockSpec((MID, MID), lambda b: (0, 0)),
            pl.BlockSpec((C, MID), lambda b: (0, 0)),
            pl.BlockSpec((C, MID), lambda b: (0, 0)),
        ],
        out_specs=[
            pl.BlockSpec((1, C, N), lambda b: (b, 0, 0)),
            pl.BlockSpec((1, C, 1), lambda b: (b, 0, 0)),
            pl.BlockSpec((1, C, 1), lambda b: (b, 0, 0)),
        ],
        out_shape=[
            jax.ShapeDtypeStruct((B, C, N), jnp.bfloat16),
            jax.ShapeDtypeStruct((B, C, 1), f32),
            jax.ShapeDtypeStruct((B, C, 1), f32),
        ],
        compiler_params=pltpu.CompilerParams(
            dimension_semantics=("parallel",)),
    )(x, curves_flat, watt2, Wa, Wb, Wc, Wn, Wl, Wd1, Wd2)

    s_cb = s[:, :, 0].T
    q_cb = q[:, :, 0].T

    out = pl.pallas_call(
        functools.partial(_pass2, count=float(B * N)),
        grid=(B,),
        in_specs=[
            pl.BlockSpec((1, C, N), lambda b: (b, 0, 0)),
            pl.BlockSpec((1, C, N), lambda b: (b, 0, 0)),
            pl.BlockSpec((C, B), lambda b: (0, 0)),
            pl.BlockSpec((C, B), lambda b: (0, 0)),
            pl.BlockSpec((C, 1), lambda b: (0, 0)),
            pl.BlockSpec((C, 1), lambda b: (0, 0)),
        ],
        out_specs=pl.BlockSpec((1, C, N), lambda b: (b, 0, 0)),
        out_shape=jax.ShapeDtypeStruct((B, C, N), f32),
        compiler_params=pltpu.CompilerParams(
            dimension_semantics=("parallel",)),
    )(d, x, s_cb, q_cb, gamma.reshape(C, 1), beta.reshape(C, 1))

    return out


# speedup vs baseline: 1.3509x; 1.3509x over previous
import functools

import jax
import jax.numpy as jnp
from jax import lax
from jax.experimental import pallas as pl
from jax.experimental.pallas import tpu as pltpu


def _pass1(x_ref, cf_ref, watt_ref, wa_ref, wb_ref, wc_ref, wn_ref, wl_ref,
           wd_ref, d_ref, s_ref, q_ref, *, CN, CL, MID):
    f32 = jnp.float32
    xb = x_ref[0]
    cf = cf_ref[0]
    J = CN * CL

    S = (lax.broadcasted_iota(jnp.int32, (J, CN), 0) // CL
         == lax.broadcasted_iota(jnp.int32, (J, CN), 1)).astype(f32)
    S2 = (lax.broadcasted_iota(jnp.int32, (CN, J), 1) // CL
          == lax.broadcasted_iota(jnp.int32, (CN, J), 0)).astype(f32)
    T = (lax.broadcasted_iota(jnp.int32, (J, CL), 0) % CL
         == lax.broadcasted_iota(jnp.int32, (J, CL), 1)).astype(f32)
    T2 = (lax.broadcasted_iota(jnp.int32, (CL, J), 1) % CL
          == lax.broadcasted_iota(jnp.int32, (CL, J), 0)).astype(f32)

    att = jnp.dot(watt_ref[...], cf, preferred_element_type=f32, precision=lax.Precision.HIGHEST)
    e = jnp.exp(att - jnp.max(att))
    den_k = jnp.dot(e, S, preferred_element_type=f32, precision=lax.Precision.HIGHEST)
    den_l = jnp.dot(e, T, preferred_element_type=f32, precision=lax.Precision.HIGHEST)
    soft_last = e / jnp.dot(den_k, S2, preferred_element_type=f32, precision=lax.Precision.HIGHEST)
    soft_pen = e / jnp.dot(den_l, T2, preferred_element_type=f32, precision=lax.Precision.HIGHEST)

    curver_inter = jnp.dot(cf * soft_last, S, preferred_element_type=f32, precision=lax.Precision.HIGHEST)
    curves_intra = jnp.dot(cf * soft_pen, T, preferred_element_type=f32, precision=lax.Precision.HIGHEST)

    CI = jnp.dot(wa_ref[...], curver_inter, preferred_element_type=f32, precision=lax.Precision.HIGHEST)
    CLm = jnp.dot(wb_ref[...], curves_intra, preferred_element_type=f32, precision=lax.Precision.HIGHEST)
    WnCI = jnp.dot(wn_ref[...], CI, preferred_element_type=f32, precision=lax.Precision.HIGHEST)
    WlCL = jnp.dot(wl_ref[...], CLm, preferred_element_type=f32, precision=lax.Precision.HIGHEST)

    G1 = lax.dot_general(CI, wc_ref[...], (((0,), (0,)), ((), ())),
                         preferred_element_type=f32, precision=lax.Precision.HIGHEST)
    G2 = lax.dot_general(CLm, wc_ref[...], (((0,), (0,)), ((), ())),
                         preferred_element_type=f32, precision=lax.Precision.HIGHEST)
    M1 = jnp.dot(wd_ref[:, :MID], WnCI, preferred_element_type=f32, precision=lax.Precision.HIGHEST)
    M2 = jnp.dot(wd_ref[:, MID:], WlCL, preferred_element_type=f32, precision=lax.Precision.HIGHEST)

    Li = jnp.dot(G1, xb, preferred_element_type=f32, precision=lax.Precision.HIGHEST)
    Ei = jnp.exp(Li - jnp.max(Li, axis=0, keepdims=True))
    Pi = Ei * (1.0 / jnp.sum(Ei, axis=0, keepdims=True))

    Ll = jnp.dot(G2, xb, preferred_element_type=f32, precision=lax.Precision.HIGHEST)
    El = jnp.exp(Ll - jnp.max(Ll, axis=0, keepdims=True))
    Pl = El * (1.0 / jnp.sum(El, axis=0, keepdims=True))

    db = (jnp.dot(M1.astype(jnp.bfloat16), Pi.astype(jnp.bfloat16),
                  preferred_element_type=f32)
          + jnp.dot(M2.astype(jnp.bfloat16), Pl.astype(jnp.bfloat16),
                    preferred_element_type=f32))
    d_ref[0] = db.astype(jnp.bfloat16)
    s_ref[0] = jnp.sum(db, axis=1, keepdims=True)
    q_ref[0] = jnp.sum(db * db, axis=1, keepdims=True)


def _pass2(d_ref, x_ref, s_ref, q_ref, g_ref, b_ref, out_ref, *, count):
    mean = jnp.sum(s_ref[...], axis=0) / count
    var = jnp.sum(q_ref[...], axis=0) / count - mean * mean
    scale = g_ref[...] * lax.rsqrt(var + 1e-5)
    shift = b_ref[...] - mean * scale
    y = x_ref[0] + d_ref[0].astype(jnp.float32) * scale + shift
    out_ref[0] = jnp.where(y >= 0, y, 0.2 * y)


@jax.jit
def kernel(x, curves, w_att, Wa, Wb, Wc, Wn, Wl, Wd, gamma, beta):
    B, C, N = x.shape
    CN, CL = curves.shape[2], curves.shape[3]
    MID = Wa.shape[0]
    J = CN * CL
    f32 = jnp.float32

    curves_flat = curves.reshape(B, C, J)
    watt2 = w_att.reshape(1, C)

    d, s, q = pl.pallas_call(
        functools.partial(_pass1, CN=CN, CL=CL, MID=MID),
        grid=(B,),
        in_specs=[
            pl.BlockSpec((1, C, N), lambda b: (b, 0, 0)),
            pl.BlockSpec((1, C, J), lambda b: (b, 0, 0)),
            pl.BlockSpec((1, C), lambda b: (0, 0)),
            pl.BlockSpec((MID, C), lambda b: (0, 0)),
            pl.BlockSpec((MID, C), lambda b: (0, 0)),
            pl.BlockSpec((MID, C), lambda b: (0, 0)),
            pl.BlockSpec((MID, MID), lambda b: (0, 0)),
            pl.BlockSpec((MID, MID), lambda b: (0, 0)),
            pl.BlockSpec((C, 2 * MID), lambda b: (0, 0)),
        ],
        out_specs=[
            pl.BlockSpec((1, C, N), lambda b: (b, 0, 0)),
            pl.BlockSpec((1, C, 1), lambda b: (b, 0, 0)),
            pl.BlockSpec((1, C, 1), lambda b: (b, 0, 0)),
        ],
        out_shape=[
            jax.ShapeDtypeStruct((B, C, N), jnp.bfloat16),
            jax.ShapeDtypeStruct((B, C, 1), f32),
            jax.ShapeDtypeStruct((B, C, 1), f32),
        ],
        compiler_params=pltpu.CompilerParams(
            dimension_semantics=("parallel",)),
    )(x, curves_flat, watt2, Wa, Wb, Wc, Wn, Wl, Wd)

    out = pl.pallas_call(
        functools.partial(_pass2, count=float(B * N)),
        grid=(B,),
        in_specs=[
            pl.BlockSpec((1, C, N), lambda b: (b, 0, 0)),
            pl.BlockSpec((1, C, N), lambda b: (b, 0, 0)),
            pl.BlockSpec((B, C, 1), lambda b: (0, 0, 0)),
            pl.BlockSpec((B, C, 1), lambda b: (0, 0, 0)),
            pl.BlockSpec((C, 1), lambda b: (0, 0)),
            pl.BlockSpec((C, 1), lambda b: (0, 0)),
        ],
        out_specs=pl.BlockSpec((1, C, N), lambda b: (b, 0, 0)),
        out_shape=jax.ShapeDtypeStruct((B, C, N), f32),
        compiler_params=pltpu.CompilerParams(
            dimension_semantics=("parallel",)),
    )(d, x, s, q, gamma.reshape(C, 1), beta.reshape(C, 1))

    return out


# speedup vs baseline: 1.3671x; 1.0120x over previous
import functools

import jax
import jax.numpy as jnp
from jax import lax
from jax.experimental import pallas as pl
from jax.experimental.pallas import tpu as pltpu

_HI = {"preferred_element_type": jnp.float32, "precision": lax.Precision.HIGHEST}


def _pass1(x_ref, cf_ref, watt_ref, wa_ref, wb_ref, wc_ref, wn_ref, wl_ref,
           wd_ref, d_ref, s_ref, q_ref, *, CN, CL, MID, PB):
    f32 = jnp.float32
    J = CN * CL

    S = (lax.broadcasted_iota(jnp.int32, (J, CN), 0) // CL
         == lax.broadcasted_iota(jnp.int32, (J, CN), 1)).astype(f32)
    S2 = (lax.broadcasted_iota(jnp.int32, (CN, J), 1) // CL
          == lax.broadcasted_iota(jnp.int32, (CN, J), 0)).astype(f32)
    T = (lax.broadcasted_iota(jnp.int32, (J, CL), 0) % CL
         == lax.broadcasted_iota(jnp.int32, (J, CL), 1)).astype(f32)
    T2 = (lax.broadcasted_iota(jnp.int32, (CL, J), 1) % CL
          == lax.broadcasted_iota(jnp.int32, (CL, J), 0)).astype(f32)

    for i in range(PB):
        xb = x_ref[i]
        cf = cf_ref[i]

        att = jnp.dot(watt_ref[...], cf, **_HI)
        e = jnp.exp(att - jnp.max(att))
        den_k = jnp.dot(e, S, **_HI)
        den_l = jnp.dot(e, T, **_HI)
        soft_last = e / jnp.dot(den_k, S2, **_HI)
        soft_pen = e / jnp.dot(den_l, T2, **_HI)

        curver_inter = jnp.dot(cf * soft_last, S, **_HI)
        curves_intra = jnp.dot(cf * soft_pen, T, **_HI)

        CI = jnp.dot(wa_ref[...], curver_inter, **_HI)
        CLm = jnp.dot(wb_ref[...], curves_intra, **_HI)
        WnCI = jnp.dot(wn_ref[...], CI, **_HI)
        WlCL = jnp.dot(wl_ref[...], CLm, **_HI)

        G1 = lax.dot_general(CI, wc_ref[...], (((0,), (0,)), ((), ())), **_HI)
        G2 = lax.dot_general(CLm, wc_ref[...], (((0,), (0,)), ((), ())), **_HI)
        M1 = jnp.dot(wd_ref[:, :MID], WnCI, **_HI)
        M2 = jnp.dot(wd_ref[:, MID:], WlCL, **_HI)

        Li = jnp.dot(G1, xb, **_HI)
        Ei = jnp.exp(Li - jnp.max(Li, axis=0, keepdims=True))
        Pi = Ei * (1.0 / jnp.sum(Ei, axis=0, keepdims=True))

        Ll = jnp.dot(G2, xb, **_HI)
        El = jnp.exp(Ll - jnp.max(Ll, axis=0, keepdims=True))
        Pl = El * (1.0 / jnp.sum(El, axis=0, keepdims=True))

        db = (jnp.dot(M1.astype(jnp.bfloat16), Pi.astype(jnp.bfloat16),
                      preferred_element_type=f32)
              + jnp.dot(M2.astype(jnp.bfloat16), Pl.astype(jnp.bfloat16),
                        preferred_element_type=f32))
        d_ref[i] = db.astype(jnp.bfloat16)
        s_ref[i] = jnp.sum(db, axis=1, keepdims=True)
        q_ref[i] = jnp.sum(db * db, axis=1, keepdims=True)


def _pass2(d_ref, x_ref, s_ref, q_ref, g_ref, b_ref, out_ref, *, count, PB):
    mean = jnp.sum(s_ref[...], axis=0) / count
    var = jnp.sum(q_ref[...], axis=0) / count - mean * mean
    scale = g_ref[...] * lax.rsqrt(var + 1e-5)
    shift = b_ref[...] - mean * scale
    for i in range(PB):
        y = x_ref[i] + d_ref[i].astype(jnp.float32) * scale + shift
        out_ref[i] = jnp.where(y >= 0, y, 0.2 * y)


@jax.jit
def kernel(x, curves, w_att, Wa, Wb, Wc, Wn, Wl, Wd, gamma, beta):
    B, C, N = x.shape
    CN, CL = curves.shape[2], curves.shape[3]
    MID = Wa.shape[0]
    J = CN * CL
    f32 = jnp.float32
    PB = 2

    curves_flat = curves.reshape(B, C, J)
    watt2 = w_att.reshape(1, C)

    d, s, q = pl.pallas_call(
        functools.partial(_pass1, CN=CN, CL=CL, MID=MID, PB=PB),
        grid=(B // PB,),
        in_specs=[
            pl.BlockSpec((PB, C, N), lambda b: (b, 0, 0)),
            pl.BlockSpec((PB, C, J), lambda b: (b, 0, 0)),
            pl.BlockSpec((1, C), lambda b: (0, 0)),
            pl.BlockSpec((MID, C), lambda b: (0, 0)),
            pl.BlockSpec((MID, C), lambda b: (0, 0)),
            pl.BlockSpec((MID, C), lambda b: (0, 0)),
            pl.BlockSpec((MID, MID), lambda b: (0, 0)),
            pl.BlockSpec((MID, MID), lambda b: (0, 0)),
            pl.BlockSpec((C, 2 * MID), lambda b: (0, 0)),
        ],
        out_specs=[
            pl.BlockSpec((PB, C, N), lambda b: (b, 0, 0)),
            pl.BlockSpec((PB, C, 1), lambda b: (b, 0, 0)),
            pl.BlockSpec((PB, C, 1), lambda b: (b, 0, 0)),
        ],
        out_shape=[
            jax.ShapeDtypeStruct((B, C, N), jnp.bfloat16),
            jax.ShapeDtypeStruct((B, C, 1), f32),
            jax.ShapeDtypeStruct((B, C, 1), f32),
        ],
        compiler_params=pltpu.CompilerParams(
            dimension_semantics=("parallel",)),
    )(x, curves_flat, watt2, Wa, Wb, Wc, Wn, Wl, Wd)

    out = pl.pallas_call(
        functools.partial(_pass2, count=float(B * N), PB=PB),
        grid=(B // PB,),
        in_specs=[
            pl.BlockSpec((PB, C, N), lambda b: (b, 0, 0)),
            pl.BlockSpec((PB, C, N), lambda b: (b, 0, 0)),
            pl.BlockSpec((B, C, 1), lambda b: (0, 0, 0)),
            pl.BlockSpec((B, C, 1), lambda b: (0, 0, 0)),
            pl.BlockSpec((C, 1), lambda b: (0, 0)),
            pl.BlockSpec((C, 1), lambda b: (0, 0)),
        ],
        out_specs=pl.BlockSpec((PB, C, N), lambda b: (b, 0, 0)),
        out_shape=jax.ShapeDtypeStruct((B, C, N), f32),
        compiler_params=pltpu.CompilerParams(
            dimension_semantics=("parallel",)),
    )(d, x, s, q, gamma.reshape(C, 1), beta.reshape(C, 1))

    return out


# speedup vs baseline: 1.6209x; 1.1856x over previous
import functools

import jax
import jax.numpy as jnp
from jax import lax
from jax.experimental import pallas as pl
from jax.experimental.pallas import tpu as pltpu

_HI = {"preferred_element_type": jnp.float32, "precision": lax.Precision.HIGHEST}


def _pass1(x_ref, cf_ref, watt_ref, wa_ref, wb_ref, wc_ref, wn_ref, wl_ref,
           wd_ref, d_ref, s_ref, q_ref, *, CN, CL, MID, PB):
    f32 = jnp.float32
    J = CN * CL

    CNP = 128
    jj = lax.broadcasted_iota(jnp.int32, (J, CNP + CL), 0)
    kk = lax.broadcasted_iota(jnp.int32, (J, CNP + CL), 1)
    ST = (((kk < CNP) & (jj // CL == kk))
          | ((kk >= CNP) & (jj % CL == kk - CNP))).astype(f32)

    for i in range(PB):
        xb = x_ref[i]
        cf = cf_ref[i]

        att = jnp.dot(watt_ref[...], cf, **_HI)
        e = jnp.exp(att - jnp.max(att))
        dens = jnp.dot(e, ST, **_HI)
        U = jnp.dot(cf * e, ST, **_HI)
        curver_inter = U[:, :CN] * (1.0 / dens[:, :CN])
        curves_intra = U[:, CNP:] * (1.0 / dens[:, CNP:])

        CI = jnp.dot(wa_ref[...], curver_inter, **_HI)
        CLm = jnp.dot(wb_ref[...], curves_intra, **_HI)
        WnCI = jnp.dot(wn_ref[...], CI, **_HI)
        WlCL = jnp.dot(wl_ref[...], CLm, **_HI)

        CIcat = jnp.concatenate(
            [CI, jnp.zeros((MID, 104 - CN), f32), CLm], axis=1)
        G12 = lax.dot_general(CIcat, wc_ref[...], (((0,), (0,)), ((), ())), **_HI)
        M1 = jnp.dot(wd_ref[:, :MID], WnCI, **_HI)
        M2 = jnp.dot(wd_ref[:, MID:], WlCL, **_HI)

        Lall = jnp.dot(G12, xb, **_HI)
        Li = Lall[:CN]
        Ll = Lall[104:]
        Ei = jnp.exp(Li - jnp.max(Li, axis=0, keepdims=True))
        Pi = Ei * (1.0 / jnp.sum(Ei, axis=0, keepdims=True))
        El = jnp.exp(Ll - jnp.max(Ll, axis=0, keepdims=True))
        Pl = El * (1.0 / jnp.sum(El, axis=0, keepdims=True))

        db = (jnp.dot(M1.astype(jnp.bfloat16), Pi.astype(jnp.bfloat16),
                      preferred_element_type=f32)
              + jnp.dot(M2.astype(jnp.bfloat16), Pl.astype(jnp.bfloat16),
                        preferred_element_type=f32))
        d_ref[i] = db.astype(jnp.bfloat16)
        s_ref[i] = jnp.sum(db, axis=1, keepdims=True)
        q_ref[i] = jnp.sum(db * db, axis=1, keepdims=True)


def _pass2(d_ref, x_ref, s_ref, q_ref, g_ref, b_ref, out_ref, *, count, PB):
    mean = jnp.sum(s_ref[...], axis=0) / count
    var = jnp.sum(q_ref[...], axis=0) / count - mean * mean
    scale = g_ref[...] * lax.rsqrt(var + 1e-5)
    shift = b_ref[...] - mean * scale
    for i in range(PB):
        y = x_ref[i] + d_ref[i].astype(jnp.float32) * scale + shift
        out_ref[i] = jnp.where(y >= 0, y, 0.2 * y)


@jax.jit
def kernel(x, curves, w_att, Wa, Wb, Wc, Wn, Wl, Wd, gamma, beta):
    B, C, N = x.shape
    CN, CL = curves.shape[2], curves.shape[3]
    MID = Wa.shape[0]
    J = CN * CL
    f32 = jnp.float32
    PB = 2

    curves_flat = curves.reshape(B, C, J)
    watt2 = w_att.reshape(1, C)

    d, s, q = pl.pallas_call(
        functools.partial(_pass1, CN=CN, CL=CL, MID=MID, PB=PB),
        grid=(B // PB,),
        in_specs=[
            pl.BlockSpec((PB, C, N), lambda b: (b, 0, 0)),
            pl.BlockSpec((PB, C, J), lambda b: (b, 0, 0)),
            pl.BlockSpec((1, C), lambda b: (0, 0)),
            pl.BlockSpec((MID, C), lambda b: (0, 0)),
            pl.BlockSpec((MID, C), lambda b: (0, 0)),
            pl.BlockSpec((MID, C), lambda b: (0, 0)),
            pl.BlockSpec((MID, MID), lambda b: (0, 0)),
            pl.BlockSpec((MID, MID), lambda b: (0, 0)),
            pl.BlockSpec((C, 2 * MID), lambda b: (0, 0)),
        ],
        out_specs=[
            pl.BlockSpec((PB, C, N), lambda b: (b, 0, 0)),
            pl.BlockSpec((PB, C, 1), lambda b: (b, 0, 0)),
            pl.BlockSpec((PB, C, 1), lambda b: (b, 0, 0)),
        ],
        out_shape=[
            jax.ShapeDtypeStruct((B, C, N), jnp.bfloat16),
            jax.ShapeDtypeStruct((B, C, 1), f32),
            jax.ShapeDtypeStruct((B, C, 1), f32),
        ],
        compiler_params=pltpu.CompilerParams(
            dimension_semantics=("parallel",)),
    )(x, curves_flat, watt2, Wa, Wb, Wc, Wn, Wl, Wd)

    out = pl.pallas_call(
        functools.partial(_pass2, count=float(B * N), PB=PB),
        grid=(B // PB,),
        in_specs=[
            pl.BlockSpec((PB, C, N), lambda b: (b, 0, 0)),
            pl.BlockSpec((PB, C, N), lambda b: (b, 0, 0)),
            pl.BlockSpec((B, C, 1), lambda b: (0, 0, 0)),
            pl.BlockSpec((B, C, 1), lambda b: (0, 0, 0)),
            pl.BlockSpec((C, 1), lambda b: (0, 0)),
            pl.BlockSpec((C, 1), lambda b: (0, 0)),
        ],
        out_specs=pl.BlockSpec((PB, C, N), lambda b: (b, 0, 0)),
        out_shape=jax.ShapeDtypeStruct((B, C, N), f32),
        compiler_params=pltpu.CompilerParams(
            dimension_semantics=("parallel",)),
    )(d, x, s, q, gamma.reshape(C, 1), beta.reshape(C, 1))

    return out


# speedup vs baseline: 1.9716x; 1.2164x over previous
import functools

import jax
import jax.numpy as jnp
from jax import lax
from jax.experimental import pallas as pl
from jax.experimental.pallas import tpu as pltpu

_HI = {"preferred_element_type": jnp.float32, "precision": lax.Precision.HIGHEST}


def _pass1(x_ref, cf_ref, watt_ref, wa_ref, wb_ref, wc_ref, wn_ref, wl_ref,
           wd_ref, d_ref, s_ref, q_ref, *, CN, CL, MID, PB):
    f32 = jnp.float32
    J = CN * CL

    CNP = 128
    jj = lax.broadcasted_iota(jnp.int32, (J, CNP + CL), 0)
    kk = lax.broadcasted_iota(jnp.int32, (J, CNP + CL), 1)
    ST = (((kk < CNP) & (jj // CL == kk))
          | ((kk >= CNP) & (jj % CL == kk - CNP))).astype(f32)

    for i in range(PB):
        xb = x_ref[i]
        cf = cf_ref[i]

        att = jnp.dot(watt_ref[...], cf, **_HI)
        e = jnp.exp(att - jnp.max(att))
        dens = jnp.dot(e, ST, **_HI)
        cfe = cf * e
        cfe_hi = cfe.astype(jnp.bfloat16)
        cfe_lo = (cfe - cfe_hi.astype(f32)).astype(jnp.bfloat16)
        STb = ST.astype(jnp.bfloat16)
        U = (jnp.dot(cfe_hi, STb, preferred_element_type=f32)
             + jnp.dot(cfe_lo, STb, preferred_element_type=f32))
        curver_inter = U[:, :CN] * (1.0 / dens[:, :CN])
        curves_intra = U[:, CNP:] * (1.0 / dens[:, CNP:])

        CI = jnp.dot(wa_ref[...], curver_inter, **_HI)
        CLm = jnp.dot(wb_ref[...], curves_intra, **_HI)
        WnCI = jnp.dot(wn_ref[...], CI, **_HI)
        WlCL = jnp.dot(wl_ref[...], CLm, **_HI)

        CIcat = jnp.concatenate(
            [CI, jnp.zeros((MID, 104 - CN), f32), CLm], axis=1)
        G12 = lax.dot_general(CIcat, wc_ref[...], (((0,), (0,)), ((), ())), **_HI)
        M1 = jnp.dot(wd_ref[:, :MID], WnCI, **_HI)
        M2 = jnp.dot(wd_ref[:, MID:], WlCL, **_HI)

        Lall = jnp.dot(G12, xb, preferred_element_type=f32)
        Li = Lall[:CN]
        Ll = Lall[104:]
        Ei = jnp.exp(Li - jnp.max(Li, axis=0, keepdims=True))
        Pi = Ei * (1.0 / jnp.sum(Ei, axis=0, keepdims=True))
        El = jnp.exp(Ll - jnp.max(Ll, axis=0, keepdims=True))
        Pl = El * (1.0 / jnp.sum(El, axis=0, keepdims=True))

        db = (jnp.dot(M1.astype(jnp.bfloat16), Pi.astype(jnp.bfloat16),
                      preferred_element_type=f32)
              + jnp.dot(M2.astype(jnp.bfloat16), Pl.astype(jnp.bfloat16),
                        preferred_element_type=f32))
        d_ref[i] = db.astype(jnp.bfloat16)
        s_ref[i] = jnp.sum(db, axis=1, keepdims=True)
        q_ref[i] = jnp.sum(db * db, axis=1, keepdims=True)


def _pass2(d_ref, x_ref, s_ref, q_ref, g_ref, b_ref, out_ref, *, count, PB):
    mean = jnp.sum(s_ref[...], axis=0) / count
    var = jnp.sum(q_ref[...], axis=0) / count - mean * mean
    scale = g_ref[...] * lax.rsqrt(var + 1e-5)
    shift = b_ref[...] - mean * scale
    for i in range(PB):
        y = x_ref[i] + d_ref[i].astype(jnp.float32) * scale + shift
        out_ref[i] = jnp.where(y >= 0, y, 0.2 * y)


@jax.jit
def kernel(x, curves, w_att, Wa, Wb, Wc, Wn, Wl, Wd, gamma, beta):
    B, C, N = x.shape
    CN, CL = curves.shape[2], curves.shape[3]
    MID = Wa.shape[0]
    J = CN * CL
    f32 = jnp.float32
    PB = 2

    curves_flat = curves.reshape(B, C, J)
    watt2 = w_att.reshape(1, C)

    d, s, q = pl.pallas_call(
        functools.partial(_pass1, CN=CN, CL=CL, MID=MID, PB=PB),
        grid=(B // PB,),
        in_specs=[
            pl.BlockSpec((PB, C, N), lambda b: (b, 0, 0)),
            pl.BlockSpec((PB, C, J), lambda b: (b, 0, 0)),
            pl.BlockSpec((1, C), lambda b: (0, 0)),
            pl.BlockSpec((MID, C), lambda b: (0, 0)),
            pl.BlockSpec((MID, C), lambda b: (0, 0)),
            pl.BlockSpec((MID, C), lambda b: (0, 0)),
            pl.BlockSpec((MID, MID), lambda b: (0, 0)),
            pl.BlockSpec((MID, MID), lambda b: (0, 0)),
            pl.BlockSpec((C, 2 * MID), lambda b: (0, 0)),
        ],
        out_specs=[
            pl.BlockSpec((PB, C, N), lambda b: (b, 0, 0)),
            pl.BlockSpec((PB, C, 1), lambda b: (b, 0, 0)),
            pl.BlockSpec((PB, C, 1), lambda b: (b, 0, 0)),
        ],
        out_shape=[
            jax.ShapeDtypeStruct((B, C, N), jnp.bfloat16),
            jax.ShapeDtypeStruct((B, C, 1), f32),
            jax.ShapeDtypeStruct((B, C, 1), f32),
        ],
        compiler_params=pltpu.CompilerParams(
            dimension_semantics=("parallel",)),
    )(x, curves_flat, watt2, Wa, Wb, Wc, Wn, Wl, Wd)

    out = pl.pallas_call(
        functools.partial(_pass2, count=float(B * N), PB=PB),
        grid=(B // PB,),
        in_specs=[
            pl.BlockSpec((PB, C, N), lambda b: (b, 0, 0)),
            pl.BlockSpec((PB, C, N), lambda b: (b, 0, 0)),
            pl.BlockSpec((B, C, 1), lambda b: (0, 0, 0)),
            pl.BlockSpec((B, C, 1), lambda b: (0, 0, 0)),
            pl.BlockSpec((C, 1), lambda b: (0, 0)),
            pl.BlockSpec((C, 1), lambda b: (0, 0)),
        ],
        out_specs=pl.BlockSpec((PB, C, N), lambda b: (b, 0, 0)),
        out_shape=jax.ShapeDtypeStruct((B, C, N), f32),
        compiler_params=pltpu.CompilerParams(
            dimension_semantics=("parallel",)),
    )(d, x, s, q, gamma.reshape(C, 1), beta.reshape(C, 1))

    return out


# speedup vs baseline: 2.1646x; 1.0979x over previous
import functools

import jax
import jax.numpy as jnp
from jax import lax
from jax.experimental import pallas as pl
from jax.experimental.pallas import tpu as pltpu

_HI = {"preferred_element_type": jnp.float32, "precision": lax.Precision.HIGHEST}


def _cic(x_ref, cf_ref, watt_ref, wa_ref, wb_ref, wc_ref, wn_ref, wl_ref,
         wd_ref, g_ref, b_ref, out_ref, d_scr, s_scr, q_scr,
         *, CN, CL, MID, PB, count):
    f32 = jnp.float32
    bf16 = jnp.bfloat16
    J = CN * CL
    ph = pl.program_id(0)
    bstep = pl.program_id(1)

    @pl.when(ph == 0)
    def _phase0():
        CNP = 128
        jj = lax.broadcasted_iota(jnp.int32, (J, CNP + CL), 0)
        kk = lax.broadcasted_iota(jnp.int32, (J, CNP + CL), 1)
        ST = (((kk < CNP) & (jj // CL == kk))
              | ((kk >= CNP) & (jj % CL == kk - CNP))).astype(f32)
        STb = ST.astype(bf16)

        ssum = jnp.zeros((wd_ref.shape[0], 1), f32)
        qsum = jnp.zeros((wd_ref.shape[0], 1), f32)
        for i in range(PB):
            xb = x_ref[i]
            cf = cf_ref[i]

            att = jnp.dot(watt_ref[...], cf, **_HI)
            e = jnp.exp(att - jnp.max(att))
            dens = jnp.dot(e, ST, **_HI)
            cfe = cf * e
            cfe_hi = cfe.astype(bf16)
            cfe_lo = (cfe - cfe_hi.astype(f32)).astype(bf16)
            U = (jnp.dot(cfe_hi, STb, preferred_element_type=f32)
                 + jnp.dot(cfe_lo, STb, preferred_element_type=f32))
            curver_inter = U[:, :CN] * (1.0 / dens[:, :CN])
            curves_intra = U[:, CNP:] * (1.0 / dens[:, CNP:])

            CI = jnp.dot(wa_ref[...], curver_inter, **_HI)
            CLm = jnp.dot(wb_ref[...], curves_intra, **_HI)
            WnCI = jnp.dot(wn_ref[...], CI, **_HI)
            WlCL = jnp.dot(wl_ref[...], CLm, **_HI)

            CIcat = jnp.concatenate(
                [CI, jnp.zeros((MID, 104 - CN), f32), CLm], axis=1)
            G12 = lax.dot_general(CIcat, wc_ref[...],
                                  (((0,), (0,)), ((), ())), **_HI)
            M1 = jnp.dot(wd_ref[:, :MID], WnCI, **_HI)
            M2 = jnp.dot(wd_ref[:, MID:], WlCL, **_HI)

            Lall = jnp.dot(G12, xb, preferred_element_type=f32)
            Li = Lall[:CN]
            Ll = Lall[104:]
            Ei = jnp.exp(Li - jnp.max(Li, axis=0, keepdims=True))
            Pi = Ei * (1.0 / jnp.sum(Ei, axis=0, keepdims=True))
            El = jnp.exp(Ll - jnp.max(Ll, axis=0, keepdims=True))
            Pl = El * (1.0 / jnp.sum(El, axis=0, keepdims=True))

            db = (jnp.dot(M1.astype(bf16), Pi.astype(bf16),
                          preferred_element_type=f32)
                  + jnp.dot(M2.astype(bf16), Pl.astype(bf16),
                            preferred_element_type=f32))
            d_scr[bstep * PB + i] = db.astype(bf16)
            ssum = ssum + jnp.sum(db, axis=1, keepdims=True)
            qsum = qsum + jnp.sum(db * db, axis=1, keepdims=True)

        @pl.when(bstep == 0)
        def _():
            s_scr[...] = ssum
            q_scr[...] = qsum

        @pl.when(bstep > 0)
        def _():
            s_scr[...] = s_scr[...] + ssum
            q_scr[...] = q_scr[...] + qsum

    @pl.when(ph == 1)
    def _phase1():
        mean = s_scr[...] / count
        var = q_scr[...] / count - mean * mean
        scale = g_ref[...] * lax.rsqrt(var + 1e-5)
        shift = b_ref[...] - mean * scale
        for i in range(PB):
            y = x_ref[i] + d_scr[bstep * PB + i].astype(jnp.float32) * scale + shift
            out_ref[i] = jnp.where(y >= 0, y, 0.2 * y)


@jax.jit
def kernel(x, curves, w_att, Wa, Wb, Wc, Wn, Wl, Wd, gamma, beta):
    B, C, N = x.shape
    CN, CL = curves.shape[2], curves.shape[3]
    MID = Wa.shape[0]
    J = CN * CL
    f32 = jnp.float32
    PB = 2

    curves_flat = curves.reshape(B, C, J)
    watt2 = w_att.reshape(1, C)

    out = pl.pallas_call(
        functools.partial(_cic, CN=CN, CL=CL, MID=MID, PB=PB,
                          count=float(B * N)),
        grid=(2, B // PB),
        in_specs=[
            pl.BlockSpec((PB, C, N), lambda p, b: (b, 0, 0)),
            pl.BlockSpec((PB, C, J), lambda p, b: (b * (1 - p), 0, 0)),
            pl.BlockSpec((1, C), lambda p, b: (0, 0)),
            pl.BlockSpec((MID, C), lambda p, b: (0, 0)),
            pl.BlockSpec((MID, C), lambda p, b: (0, 0)),
            pl.BlockSpec((MID, C), lambda p, b: (0, 0)),
            pl.BlockSpec((MID, MID), lambda p, b: (0, 0)),
            pl.BlockSpec((MID, MID), lambda p, b: (0, 0)),
            pl.BlockSpec((C, 2 * MID), lambda p, b: (0, 0)),
            pl.BlockSpec((C, 1), lambda p, b: (0, 0)),
            pl.BlockSpec((C, 1), lambda p, b: (0, 0)),
        ],
        out_specs=pl.BlockSpec((PB, C, N), lambda p, b: (b, 0, 0)),
        out_shape=jax.ShapeDtypeStruct((B, C, N), f32),
        scratch_shapes=[
            pltpu.VMEM((B, C, N), jnp.bfloat16),
            pltpu.VMEM((C, 1), f32),
            pltpu.VMEM((C, 1), f32),
        ],
    )(x, curves_flat, watt2, Wa, Wb, Wc, Wn, Wl, Wd,
      gamma.reshape(C, 1), beta.reshape(C, 1))

    return out


# speedup vs baseline: 2.2070x; 1.0196x over previous
import functools

import jax
import jax.numpy as jnp
from jax import lax
from jax.experimental import pallas as pl
from jax.experimental.pallas import tpu as pltpu

_HI = {"preferred_element_type": jnp.float32, "precision": lax.Precision.HIGHEST}


def _cic(x_ref, cf_ref, watt_ref, wa_ref, wb_ref, wc_ref, wn_ref, wl_ref,
         wd_ref, g_ref, b_ref, out_ref, d_scr, s_scr, q_scr,
         *, CN, CL, MID, PB, count):
    f32 = jnp.float32
    bf16 = jnp.bfloat16
    J = CN * CL
    ph = pl.program_id(0)
    bstep = pl.program_id(1)

    @pl.when(ph == 0)
    def _phase0():
        CNP = 128
        jj = lax.broadcasted_iota(jnp.int32, (J, CNP + CL), 0)
        kk = lax.broadcasted_iota(jnp.int32, (J, CNP + CL), 1)
        ST = (((kk < CNP) & (jj // CL == kk))
              | ((kk >= CNP) & (jj % CL == kk - CNP))).astype(f32)
        STb = ST.astype(bf16)

        wa = wa_ref[...]
        wa_hi = wa.astype(bf16)
        wa_lo = (wa - wa_hi.astype(f32)).astype(bf16)
        wb = wb_ref[...]
        wb_hi = wb.astype(bf16)
        wb_lo = (wb - wb_hi.astype(f32)).astype(bf16)
        wc = wc_ref[...]
        wc_hi = wc.astype(bf16)
        wc_lo = (wc - wc_hi.astype(f32)).astype(bf16)

        def _split(a):
            a_hi = a.astype(bf16)
            return a_hi, (a - a_hi.astype(f32)).astype(bf16)

        def _dot3(a_hi, a_lo, b_hi, b_lo):
            return (jnp.dot(a_hi, b_hi, preferred_element_type=f32)
                    + jnp.dot(a_hi, b_lo, preferred_element_type=f32)
                    + jnp.dot(a_lo, b_hi, preferred_element_type=f32))

        ssum = jnp.zeros((wd_ref.shape[0], 1), f32)
        qsum = jnp.zeros((wd_ref.shape[0], 1), f32)
        for i in range(PB):
            xb = x_ref[i]
            cf = cf_ref[i]

            att = jnp.dot(watt_ref[...], cf, **_HI)
            e = jnp.exp(att - jnp.max(att))
            dens = jnp.dot(e, ST, **_HI)
            cfe = cf * e
            cfe_hi = cfe.astype(bf16)
            cfe_lo = (cfe - cfe_hi.astype(f32)).astype(bf16)
            U = (jnp.dot(cfe_hi, STb, preferred_element_type=f32)
                 + jnp.dot(cfe_lo, STb, preferred_element_type=f32))
            curver_inter = U[:, :CN] * (1.0 / dens[:, :CN])
            curves_intra = U[:, CNP:] * (1.0 / dens[:, CNP:])

            ci_hi, ci_lo = _split(curver_inter)
            CI = _dot3(wa_hi, wa_lo, ci_hi, ci_lo)
            cl_hi, cl_lo = _split(curves_intra)
            CLm = _dot3(wb_hi, wb_lo, cl_hi, cl_lo)
            WnCI = jnp.dot(wn_ref[...], CI, **_HI)
            WlCL = jnp.dot(wl_ref[...], CLm, **_HI)

            CIcat = jnp.concatenate(
                [CI, jnp.zeros((MID, 104 - CN), f32), CLm], axis=1)
            cc_hi, cc_lo = _split(CIcat)
            dg = (((0,), (0,)), ((), ()))
            G12 = (lax.dot_general(cc_hi, wc_hi, dg, preferred_element_type=f32)
                   + lax.dot_general(cc_hi, wc_lo, dg, preferred_element_type=f32)
                   + lax.dot_general(cc_lo, wc_hi, dg, preferred_element_type=f32))
            M1 = jnp.dot(wd_ref[:, :MID], WnCI, **_HI)
            M2 = jnp.dot(wd_ref[:, MID:], WlCL, **_HI)

            Lall = jnp.dot(G12, xb, preferred_element_type=f32)
            Li = Lall[:CN]
            Ll = Lall[104:]
            Ei = jnp.exp(Li - jnp.max(Li, axis=0, keepdims=True))
            Pi = Ei * (1.0 / jnp.sum(Ei, axis=0, keepdims=True))
            El = jnp.exp(Ll - jnp.max(Ll, axis=0, keepdims=True))
            Pl = El * (1.0 / jnp.sum(El, axis=0, keepdims=True))

            db = (jnp.dot(M1.astype(bf16), Pi.astype(bf16),
                          preferred_element_type=f32)
                  + jnp.dot(M2.astype(bf16), Pl.astype(bf16),
                            preferred_element_type=f32))
            d_scr[bstep * PB + i] = db.astype(bf16)
            ssum = ssum + jnp.sum(db, axis=1, keepdims=True)
            qsum = qsum + jnp.sum(db * db, axis=1, keepdims=True)

        @pl.when(bstep == 0)
        def _():
            s_scr[...] = ssum
            q_scr[...] = qsum

        @pl.when(bstep > 0)
        def _():
            s_scr[...] = s_scr[...] + ssum
            q_scr[...] = q_scr[...] + qsum

    @pl.when(ph == 1)
    def _phase1():
        mean = s_scr[...] / count
        var = q_scr[...] / count - mean * mean
        scale = g_ref[...] * lax.rsqrt(var + 1e-5)
        shift = b_ref[...] - mean * scale
        for i in range(PB):
            y = x_ref[i] + d_scr[bstep * PB + i].astype(jnp.float32) * scale + shift
            out_ref[i] = jnp.where(y >= 0, y, 0.2 * y)


@jax.jit
def kernel(x, curves, w_att, Wa, Wb, Wc, Wn, Wl, Wd, gamma, beta):
    B, C, N = x.shape
    CN, CL = curves.shape[2], curves.shape[3]
    MID = Wa.shape[0]
    J = CN * CL
    f32 = jnp.float32
    PB = 2

    curves_flat = curves.reshape(B, C, J)
    watt2 = w_att.reshape(1, C)

    out = pl.pallas_call(
        functools.partial(_cic, CN=CN, CL=CL, MID=MID, PB=PB,
                          count=float(B * N)),
        grid=(2, B // PB),
        in_specs=[
            pl.BlockSpec((PB, C, N), lambda p, b: (b, 0, 0)),
            pl.BlockSpec((PB, C, J), lambda p, b: (b * (1 - p), 0, 0)),
            pl.BlockSpec((1, C), lambda p, b: (0, 0)),
            pl.BlockSpec((MID, C), lambda p, b: (0, 0)),
            pl.BlockSpec((MID, C), lambda p, b: (0, 0)),
            pl.BlockSpec((MID, C), lambda p, b: (0, 0)),
            pl.BlockSpec((MID, MID), lambda p, b: (0, 0)),
            pl.BlockSpec((MID, MID), lambda p, b: (0, 0)),
            pl.BlockSpec((C, 2 * MID), lambda p, b: (0, 0)),
            pl.BlockSpec((C, 1), lambda p, b: (0, 0)),
            pl.BlockSpec((C, 1), lambda p, b: (0, 0)),
        ],
        out_specs=pl.BlockSpec((PB, C, N), lambda p, b: (b, 0, 0)),
        out_shape=jax.ShapeDtypeStruct((B, C, N), f32),
        scratch_shapes=[
            pltpu.VMEM((B, C, N), jnp.bfloat16),
            pltpu.VMEM((C, 1), f32),
            pltpu.VMEM((C, 1), f32),
        ],
    )(x, curves_flat, watt2, Wa, Wb, Wc, Wn, Wl, Wd,
      gamma.reshape(C, 1), beta.reshape(C, 1))

    return out
